# baseline (speedup 1.0000x reference)
import functools

import jax
import jax.numpy as jnp
from jax import lax
from jax.experimental import pallas as pl
from jax.experimental.pallas import tpu as pltpu
from jax.experimental.pallas import tpu_sc as plsc

FEAT_DIM = 512
NUM_IDS = 100000
B = 4096
L = 16
CHUNK = 32
G_SPLIT = 2


@functools.cache
def _build_sc():
    info = plsc.get_sparse_core_info()
    nw = info.num_cores * info.num_subcores
    b_per_w = B // nw
    n_chunks = b_per_w // CHUNK
    g_rows = b_per_w // G_SPLIT
    vecs_per_row = FEAT_DIM // L

    mesh = plsc.VectorSubcoreMesh(core_axis_name="c", subcore_axis_name="s")

    @functools.partial(
        pl.kernel,
        mesh=mesh,
        out_type=jax.ShapeDtypeStruct((B, FEAT_DIM), jnp.float32),
        scratch_types=(
            [pltpu.VMEM((b_per_w,), jnp.int32),
             pltpu.VMEM((b_per_w, FEAT_DIM), jnp.float32),
             pltpu.VMEM((CHUNK, FEAT_DIM), jnp.float32),
             pltpu.VMEM((CHUNK, FEAT_DIM), jnp.float32)]
            + [pltpu.SemaphoreType.DMA for _ in range(G_SPLIT + 2 + n_chunks)]
        ),
    )
    def k(table_hbm, idx_hbm, feat_hbm, out_hbm, idx_v, rows_v, feat0, feat1,
          *sems):
        gsems = sems[:G_SPLIT]
        fsems = sems[G_SPLIT:G_SPLIT + 2]
        osems = sems[G_SPLIT + 2:]
        isem = osems[-1]
        feats = (feat0, feat1)

        wid = lax.axis_index("s") * info.num_cores + lax.axis_index("c")
        base = wid * b_per_w

        feat_cp = [None] * n_chunks
        for c in range(2):
            feat_cp[c] = pltpu.async_copy(
                feat_hbm.at[pl.ds(base + c * CHUNK, CHUNK)], feats[c], fsems[c])
        pltpu.async_copy(idx_hbm.at[pl.ds(base, b_per_w)], idx_v, isem).wait()
        gather_cp = [
            pltpu.async_copy(
                table_hbm.at[idx_v.at[pl.ds(g * g_rows, g_rows)]],
                rows_v.at[pl.ds(g * g_rows, g_rows)], gsems[g])
            for g in range(G_SPLIT)
        ]

        out_cp = [None] * n_chunks
        chunks_per_g = g_rows // CHUNK
        for c in range(n_chunks):
            b = c % 2
            if c % chunks_per_g == 0:
                gather_cp[c // chunks_per_g].wait()
            feat_cp[c].wait()
            fb = feats[b]
            row0 = c * CHUNK

            def blend_row(i, carry):
                for v in range(vecs_per_row):
                    sl = pl.ds(v * L, L)
                    rows_v[row0 + i, sl] = (
                        0.7 * fb[i, sl] + 0.3 * rows_v[row0 + i, sl])
                return carry

            lax.fori_loop(0, CHUNK, blend_row, 0)
            if c + 2 < n_chunks:
                feat_cp[c + 2] = pltpu.async_copy(
                    feat_hbm.at[pl.ds(base + (c + 2) * CHUNK, CHUNK)],
                    feats[b], fsems[b])
            out_cp[c] = pltpu.async_copy(
                rows_v.at[pl.ds(row0, CHUNK)],
                out_hbm.at[pl.ds(base + row0, CHUNK)], osems[c])
        for c in range(n_chunks):
            out_cp[c].wait()

    return k


def kernel(feat, modality, pids, global_proto, local_proto):
    del local_proto
    table = global_proto.reshape(2 * NUM_IDS, FEAT_DIM)
    flat_idx = modality * NUM_IDS + pids
    return _build_sc()(table, flat_idx, feat)
